# baseline (speedup 1.0000x reference)
import jax
import jax.numpy as jnp
from jax import lax
from jax.experimental import pallas as pl
from jax.experimental.pallas import tpu as pltpu
from jax.experimental.pallas import tpu_sc as plsc

VOCAB = 100000
B = 4096
HIST = 50
D = 64
U = 128
ROW_W = 128

NC = 2
NS = 16
NW = NC * NS
ROWS_PER_W = B // NW
NBUF = 4

TCB = 512


def _sq_body(xt_ref, o_ref):
  o_ref[:, 0:D] = jnp.transpose(xt_ref[...]) ** 2
  o_ref[:, D:ROW_W] = jnp.zeros((TCB, ROW_W - D), jnp.float32)


def _sq_table(emb_t):
  return pl.pallas_call(
      _sq_body,
      out_shape=jax.ShapeDtypeStruct((VOCAB, ROW_W), jnp.float32),
      grid=(pl.cdiv(VOCAB, TCB),),
      in_specs=[pl.BlockSpec((D, TCB), lambda j: (0, j))],
      out_specs=pl.BlockSpec((TCB, ROW_W), lambda j: (j, 0)),
  )(emb_t)


def _sc_pooled_body(table_hbm, ids_hbm, out_hbm, idx_v, rows_v, pooled_v,
                    *sems):
  cid = lax.axis_index("c")
  sid = lax.axis_index("s")
  wid = sid * NC + cid
  base = wid * ROWS_PER_W

  pltpu.sync_copy(ids_hbm.at[pl.ds(base, ROWS_PER_W)], idx_v)

  for b in range(NBUF):
    pltpu.make_async_copy(
        table_hbm.at[idx_v.at[b]], rows_v.at[b], sems[b]).start()

  def group_body(g, carry):
    for b in range(NBUF):
      j = g * NBUF + b
      pltpu.make_async_copy(
          table_hbm.at[idx_v.at[j]], rows_v.at[b], sems[b]).wait()
      acc = [jnp.zeros((16,), jnp.float32) for _ in range(4)]
      for l in range(HIST):
        for v in range(4):
          acc[v] = acc[v] + rows_v[b, l, pl.ds(v * 16, 16)]
      for v in range(4):
        pooled_v[j, pl.ds(v * 16, 16)] = acc[v]

      @pl.when(j + NBUF < ROWS_PER_W)
      def _refill():
        pltpu.make_async_copy(
            table_hbm.at[idx_v.at[j + NBUF]], rows_v.at[b], sems[b]).start()
    return carry

  lax.fori_loop(0, ROWS_PER_W // NBUF, group_body, 0)

  pltpu.sync_copy(pooled_v, out_hbm.at[pl.ds(base, ROWS_PER_W)])


def _sc_pooled(sq_table, ids):
  mesh = plsc.VectorSubcoreMesh(core_axis_name="c", subcore_axis_name="s")
  return pl.kernel(
      _sc_pooled_body,
      out_type=jax.ShapeDtypeStruct((B, D), jnp.float32),
      mesh=mesh,
      compiler_params=pltpu.CompilerParams(use_tc_tiling_on_sc=False),
      scratch_types=[
          pltpu.VMEM((ROWS_PER_W, HIST), jnp.int32),
          pltpu.VMEM((NBUF, HIST, ROW_W), jnp.float32),
          pltpu.VMEM((ROWS_PER_W, D), jnp.float32),
      ] + [pltpu.SemaphoreType.DMA] * NBUF,
  )(sq_table, ids)


def _mm_body(p_ref, k_ref, o_ref):
  o_ref[...] = jnp.dot(p_ref[...], k_ref[...],
                       preferred_element_type=jnp.float32)


def _dense(pooled, w):
  return pl.pallas_call(
      _mm_body,
      out_shape=jax.ShapeDtypeStruct((B, U), jnp.float32),
  )(pooled, w)


@jax.jit
def _run(input_ids, embedding, w):
  sq = _sq_table(embedding.T)
  pooled = _sc_pooled(sq, input_ids)
  return _dense(pooled, w)


def kernel(input_ids, embedding, kernel):
  return _run(input_ids, embedding, kernel)


# speedup vs baseline: 1.3278x; 1.3278x over previous
import jax
import jax.numpy as jnp
from jax import lax
from jax.experimental import pallas as pl
from jax.experimental.pallas import tpu as pltpu
from jax.experimental.pallas import tpu_sc as plsc

VOCAB = 100000
B = 4096
HIST = 50
D = 64
U = 128
ROW_W = 128
TCB = 512
HALF = 50176

NC = 2
NS = 16
NW = NC * NS
ROWS_PER_W = B // NW
NBUF = 4


def _sq_body(lo_ref, hi_ref, o_ref):
  r = lax.broadcasted_iota(jnp.int32, (D, D), 0)
  c = lax.broadcasted_iota(jnp.int32, (D, D), 1)
  eye = (r == c).astype(jnp.float32)
  dn = (((0,), (0,)), ((), ()))
  lo = lo_ref[...]
  hi = hi_ref[...]
  o_ref[:, 0:D] = lax.dot_general(lo * lo, eye, dn,
                                  preferred_element_type=jnp.float32)
  o_ref[:, D:ROW_W] = lax.dot_general(hi * hi, eye, dn,
                                      preferred_element_type=jnp.float32)


def _sq_table(emb_t):
  return pl.pallas_call(
      _sq_body,
      out_shape=jax.ShapeDtypeStruct((HALF, ROW_W), jnp.float32),
      grid=(HALF // TCB,),
      in_specs=[
          pl.BlockSpec((D, TCB), lambda j: (0, j)),
          pl.BlockSpec((D, TCB), lambda j: (0, j + HALF // TCB)),
      ],
      out_specs=pl.BlockSpec((TCB, ROW_W), lambda j: (j, 0)),
  )(emb_t, emb_t)


_WSTARTS = (0, 16, 32, 34)


def _mod_row(idx_v, midx_v, j):
  for s in _WSTARTS:
    v = idx_v[j, pl.ds(s, 16)]
    midx_v[j, pl.ds(s, 16)] = jnp.where(v >= HALF, v - HALF, v)


def _sc_pooled_body(table_hbm, ids_hbm, out_hbm, idx_v, midx_v, rows_v,
                    pooled_v, *sems):
  cid = lax.axis_index("c")
  sid = lax.axis_index("s")
  wid = sid * NC + cid
  base = wid * ROWS_PER_W

  pltpu.sync_copy(ids_hbm.at[pl.ds(base, ROWS_PER_W)], idx_v)

  for b in range(NBUF):
    _mod_row(idx_v, midx_v, b)
    pltpu.make_async_copy(
        table_hbm.at[midx_v.at[b]], rows_v.at[b], sems[b]).start()

  def group_body(g, carry):
    for b in range(NBUF):
      j = g * NBUF + b
      pltpu.make_async_copy(
          table_hbm.at[midx_v.at[j]], rows_v.at[b], sems[b]).wait()
      hv = [idx_v[j, pl.ds(s, 16)] for s in _WSTARTS]
      acc = [jnp.zeros((16,), jnp.float32) for _ in range(4)]
      for l in range(HIST):
        w = 3 if l >= 48 else l // 16
        h = hv[w][l - _WSTARTS[w]]
        off = jnp.where(h >= HALF, D, 0)
        for v in range(4):
          acc[v] = acc[v] + rows_v[b, l, pl.ds(off + v * 16, 16)]
      for v in range(4):
        pooled_v[j, pl.ds(v * 16, 16)] = acc[v]

      @pl.when(j + NBUF < ROWS_PER_W)
      def _refill():
        _mod_row(idx_v, midx_v, j + NBUF)
        pltpu.make_async_copy(
            table_hbm.at[midx_v.at[j + NBUF]], rows_v.at[b], sems[b]).start()
    return carry

  lax.fori_loop(0, ROWS_PER_W // NBUF, group_body, 0)

  pltpu.sync_copy(pooled_v, out_hbm.at[pl.ds(base, ROWS_PER_W)])


def _sc_pooled(sq_table, ids):
  mesh = plsc.VectorSubcoreMesh(core_axis_name="c", subcore_axis_name="s")
  return pl.kernel(
      _sc_pooled_body,
      out_type=jax.ShapeDtypeStruct((B, D), jnp.float32),
      mesh=mesh,
      compiler_params=pltpu.CompilerParams(use_tc_tiling_on_sc=False),
      scratch_types=[
          pltpu.VMEM((ROWS_PER_W, HIST), jnp.int32),
          pltpu.VMEM((ROWS_PER_W, HIST), jnp.int32),
          pltpu.VMEM((NBUF, HIST, ROW_W), jnp.float32),
          pltpu.VMEM((ROWS_PER_W, D), jnp.float32),
      ] + [pltpu.SemaphoreType.DMA] * NBUF,
  )(sq_table, ids)


def _mm_body(p_ref, k_ref, o_ref):
  o_ref[...] = jnp.dot(p_ref[...], k_ref[...],
                       preferred_element_type=jnp.float32)


def _dense(pooled, w):
  return pl.pallas_call(
      _mm_body,
      out_shape=jax.ShapeDtypeStruct((B, U), jnp.float32),
  )(pooled, w)


@jax.jit
def _run(input_ids, embedding, w):
  sq = _sq_table(embedding.T)
  pooled = _sc_pooled(sq, input_ids)
  return _dense(pooled, w)


def kernel(input_ids, embedding, kernel):
  return _run(input_ids, embedding, kernel)
